# baseline (speedup 1.0000x reference)
import jax
import jax.numpy as jnp
from jax import lax
from jax.experimental import pallas as pl
from jax.experimental.pallas import tpu as pltpu
from jax.experimental.pallas import tpu_sc as plsc

B, S, D = 4, 8192, 1024
NW = 32
RPW = S // NW
R = 8
NCH = RPW // R
NTRI = 10
L = 16
UNROLL = 16
NVEC = R * D // L


def _body(x_hbm, pe_hbm, out_hbm, xb, pb, in_sem, out_sem, pe_sem):
    wid = lax.axis_index("s") * 2 + lax.axis_index("c")
    row0 = wid * RPW

    def issue_pe(g, p):
        pltpu.async_copy(pe_hbm.at[pl.ds(row0 + g * R, R), :], pb.at[p], pe_sem)

    def drain_pe(p):
        pltpu.make_async_copy(pe_hbm.at[pl.ds(0, R), :], pb.at[p], pe_sem).wait()

    def issue_in(g, p):
        pltpu.async_copy(x_hbm.at[:, pl.ds(row0 + g * R, R), :], xb.at[p], in_sem)

    def drain_in(p):
        pltpu.make_async_copy(x_hbm.at[:, pl.ds(0, R), :], xb.at[p], in_sem).wait()

    def issue_out(g, p):
        pltpu.async_copy(xb.at[p], out_hbm.at[:, pl.ds(row0 + g * R, R), :], out_sem)

    def drain_out(p):
        pltpu.make_async_copy(xb.at[p], out_hbm.at[:, pl.ds(0, R), :], out_sem).wait()

    def compute(p):
        def add_body(i, _):
            for k in range(UNROLL):
                flat = i * UNROLL + k
                r = flat // (D // L)
                c = (flat % (D // L)) * L
                sl = pl.ds(c, L)
                pe_v = pb[p, r, sl]
                for b in range(B):
                    xb[p, b, r, sl] = xb[p, b, r, sl] + pe_v
            return 0

        lax.fori_loop(0, NVEC // UNROLL, add_body, 0)

    def _maybe(cond, fn):
        if cond is True:
            fn()
        else:
            pl.when(cond)(fn)

    def chunk(g, p, drain_prev, issue_next):
        pn = (p + 1) % 3
        _maybe(drain_prev, lambda: drain_out(pn))
        _maybe(issue_next, lambda: (issue_in(g + 1, pn), issue_pe(g + 1, pn)))
        drain_in(p)
        drain_pe(p)
        compute(p)
        issue_out(g, p)

    issue_in(0, 0)
    issue_pe(0, 0)

    def tri_body(t, _):
        g = t * 3
        chunk(g, 0, t > 0, True)
        chunk(g + 1, 1, t > 0, True)
        chunk(g + 2, 2, True, True)
        return 0

    lax.fori_loop(0, NTRI, tri_body, 0)
    chunk(NCH - 2, 0, True, True)
    chunk(NCH - 1, 1, True, False)
    drain_out(0)
    drain_out(1)


def kernel(x, pe_table):
    mesh = plsc.VectorSubcoreMesh(core_axis_name="c", subcore_axis_name="s")
    run = pl.kernel(
        _body,
        mesh=mesh,
        out_type=jax.ShapeDtypeStruct((B, S, D), jnp.float32),
        scratch_types=[
            pltpu.VMEM((3, B, R, D), jnp.float32),
            pltpu.VMEM((3, R, D), jnp.float32),
            pltpu.SemaphoreType.DMA,
            pltpu.SemaphoreType.DMA,
            pltpu.SemaphoreType.DMA,
        ],
        compiler_params=pltpu.CompilerParams(use_tc_tiling_on_sc=True),
    )
    return run(x, pe_table)


# speedup vs baseline: 1.0149x; 1.0149x over previous
import jax
import jax.numpy as jnp
from jax import lax
from jax.experimental import pallas as pl
from jax.experimental.pallas import tpu as pltpu
from jax.experimental.pallas import tpu_sc as plsc

B, S, D = 4, 8192, 1024
NW = 32
RPW = S // NW
R = 8
NCH = RPW // R
NTRI = 10
L = 16
UNROLL = 8
NVEC = R * D // L


def _body(x_hbm, pe_hbm, out_hbm, xb, pb, in_sem, out_sem, pe_sem):
    wid = lax.axis_index("s") * 2 + lax.axis_index("c")
    row0 = wid * RPW

    def issue_pe(g, p):
        pltpu.async_copy(pe_hbm.at[pl.ds(row0 + g * R, R), :], pb.at[p], pe_sem)

    def drain_pe(p):
        pltpu.make_async_copy(pe_hbm.at[pl.ds(0, R), :], pb.at[p], pe_sem).wait()

    def issue_in(g, p):
        pltpu.async_copy(x_hbm.at[:, pl.ds(row0 + g * R, R), :], xb.at[p], in_sem)

    def drain_in(p):
        pltpu.make_async_copy(x_hbm.at[:, pl.ds(0, R), :], xb.at[p], in_sem).wait()

    def issue_out(g, p):
        pltpu.async_copy(xb.at[p], out_hbm.at[:, pl.ds(row0 + g * R, R), :], out_sem)

    def drain_out(p):
        pltpu.make_async_copy(xb.at[p], out_hbm.at[:, pl.ds(0, R), :], out_sem).wait()

    def compute(p):
        def add_body(i, _):
            for k in range(UNROLL):
                flat = i * UNROLL + k
                r = flat // (D // L)
                c = (flat % (D // L)) * L
                sl = pl.ds(c, L)
                pe_v = pb[p, r, sl]
                for b in range(B):
                    xb[p, b, r, sl] = xb[p, b, r, sl] + pe_v
            return 0

        lax.fori_loop(0, NVEC // UNROLL, add_body, 0)

    def _maybe(cond, fn):
        if cond is True:
            fn()
        else:
            pl.when(cond)(fn)

    def chunk(g, p, drain_prev, issue_next):
        pn = (p + 1) % 3
        drain_in(p)
        drain_pe(p)
        _maybe(drain_prev, lambda: drain_out(pn))
        _maybe(issue_next, lambda: (issue_in(g + 1, pn), issue_pe(g + 1, pn)))
        compute(p)
        issue_out(g, p)

    issue_in(0, 0)
    issue_pe(0, 0)

    def tri_body(t, _):
        g = t * 3
        chunk(g, 0, t > 0, True)
        chunk(g + 1, 1, t > 0, True)
        chunk(g + 2, 2, True, True)
        return 0

    lax.fori_loop(0, NTRI, tri_body, 0)
    chunk(NCH - 2, 0, True, True)
    chunk(NCH - 1, 1, True, False)
    drain_out(0)
    drain_out(1)


def kernel(x, pe_table):
    mesh = plsc.VectorSubcoreMesh(core_axis_name="c", subcore_axis_name="s")
    run = pl.kernel(
        _body,
        mesh=mesh,
        out_type=jax.ShapeDtypeStruct((B, S, D), jnp.float32),
        scratch_types=[
            pltpu.VMEM((3, B, R, D), jnp.float32),
            pltpu.VMEM((3, R, D), jnp.float32),
            pltpu.SemaphoreType.DMA,
            pltpu.SemaphoreType.DMA,
            pltpu.SemaphoreType.DMA,
        ],
        compiler_params=pltpu.CompilerParams(use_tc_tiling_on_sc=True),
    )
    return run(x, pe_table)


# speedup vs baseline: 1.0169x; 1.0020x over previous
import jax
import jax.numpy as jnp
from jax import lax
from jax.experimental import pallas as pl
from jax.experimental.pallas import tpu as pltpu
from jax.experimental.pallas import tpu_sc as plsc

B, S, D = 4, 8192, 1024
NW = 32
RPW = S // NW
R = 8
NCH = RPW // R
NTRI = 10
L = 16
UNROLL = 8
NVEC = R * D // L


def _body(x_hbm, pe_hbm, out_hbm, xb, pb, in_sem, out_sem, pe_sem):
    wid = lax.axis_index("s") * 2 + lax.axis_index("c")
    row0 = wid * RPW

    def issue_pe(g, p):
        pltpu.async_copy(pe_hbm.at[pl.ds(row0 + g * R, R), :], pb.at[p], pe_sem)

    def drain_pe(p):
        pltpu.make_async_copy(pe_hbm.at[pl.ds(0, R), :], pb.at[p], pe_sem).wait()

    def issue_in(g, p):
        pltpu.async_copy(x_hbm.at[:, pl.ds(row0 + g * R, R), :], xb.at[p], in_sem)

    def drain_in(p):
        pltpu.make_async_copy(x_hbm.at[:, pl.ds(0, R), :], xb.at[p], in_sem).wait()

    def issue_out(g, p):
        pltpu.async_copy(xb.at[p], out_hbm.at[:, pl.ds(row0 + g * R, R), :], out_sem)

    def drain_out(p):
        pltpu.make_async_copy(xb.at[p], out_hbm.at[:, pl.ds(0, R), :], out_sem).wait()

    def compute(p):
        def add_body(i, _):
            for k in range(UNROLL):
                flat = i * UNROLL + k
                r = flat // (D // L)
                c = (flat % (D // L)) * L
                sl = pl.ds(c, L)
                pe_v = pb[p, r, sl]
                for b in range(B):
                    xb[p, b, r, sl] = xb[p, b, r, sl] + pe_v
            return 0

        lax.fori_loop(0, NVEC // UNROLL, add_body, 0)

    def _maybe(cond, fn):
        if cond is True:
            fn()
        else:
            pl.when(cond)(fn)

    def chunk(g, p, drain_prev, issue_next):
        pn = (p + 1) % 3
        drain_in(p)
        drain_pe(p)
        _maybe(drain_prev, lambda: drain_out(pn))
        _maybe(issue_next, lambda: (issue_in(g + 1, pn), issue_pe(g + 1, pn)))
        compute(p)
        issue_out(g, p)

    issue_in(0, 0)
    issue_pe(0, 0)

    def tri_body(t, _):
        g = t * 3
        chunk(g, 0, t > 0, True)
        chunk(g + 1, 1, t > 0, True)
        chunk(g + 2, 2, True, True)
        return 0

    lax.fori_loop(0, NTRI, tri_body, 0)
    chunk(NCH - 2, 0, True, True)
    chunk(NCH - 1, 1, True, False)
    drain_out(0)
    drain_out(1)


def kernel(x, pe_table):
    mesh = plsc.VectorSubcoreMesh(core_axis_name="c", subcore_axis_name="s")
    run = pl.kernel(
        _body,
        mesh=mesh,
        out_type=jax.ShapeDtypeStruct((B, S, D), jnp.float32),
        scratch_types=[
            pltpu.VMEM((3, B, R, D), jnp.float32),
            pltpu.VMEM((3, R, D), jnp.float32),
            pltpu.SemaphoreType.DMA,
            pltpu.SemaphoreType.DMA,
            pltpu.SemaphoreType.DMA,
        ],
        compiler_params=pltpu.CompilerParams(
            use_tc_tiling_on_sc=True,
            disable_bounds_checks=True,
            disable_semaphore_checks=True,
        ),
    )
    return run(x, pe_table)


# speedup vs baseline: 1.0189x; 1.0020x over previous
import jax
import jax.numpy as jnp
from jax import lax
from jax.experimental import pallas as pl
from jax.experimental.pallas import tpu as pltpu
from jax.experimental.pallas import tpu_sc as plsc

B, S, D = 4, 8192, 1024
NW = 32
RPW = S // NW
R = 8
NCH = RPW // R
NTRI = 10
L = 16
UNROLL = 8
NVEC = R * D // L


def _body(x_hbm, pe_hbm, out_hbm, xb, pb, in_sem, out_sem, pe_sem):
    wid = lax.axis_index("s") * 2 + lax.axis_index("c")
    row0 = wid * RPW

    def issue_pe(g, p):
        pltpu.async_copy(pe_hbm.at[pl.ds(row0 + g * R, R), :], pb.at[p], pe_sem)

    def drain_pe(p):
        pltpu.make_async_copy(pe_hbm.at[pl.ds(0, R), :], pb.at[p], pe_sem).wait()

    def issue_in(g, p):
        pltpu.async_copy(x_hbm.at[:, pl.ds(row0 + g * R, R), :], xb.at[p], in_sem)

    def drain_in(p):
        pltpu.make_async_copy(x_hbm.at[:, pl.ds(0, R), :], xb.at[p], in_sem).wait()

    def issue_out(g, p):
        pltpu.async_copy(xb.at[p], out_hbm.at[:, pl.ds(row0 + g * R, R), :], out_sem)

    def drain_out(p):
        pltpu.make_async_copy(xb.at[p], out_hbm.at[:, pl.ds(0, R), :], out_sem).wait()

    def compute(p):
        def add_body(i, _):
            for k in range(UNROLL):
                flat = i * UNROLL + k
                r = flat // (D // L)
                c = (flat % (D // L)) * L
                sl = pl.ds(c, L)
                pe_v = pb[p, r, sl]
                for b in range(B):
                    xb[p, b, r, sl] = xb[p, b, r, sl] + pe_v
            return 0

        lax.fori_loop(0, NVEC // UNROLL, add_body, 0)

    def _maybe(cond, fn):
        if cond is True:
            fn()
        else:
            pl.when(cond)(fn)

    def chunk(g, p, drain_prev, issue_next):
        pn = (p + 1) % 3
        drain_in(p)
        drain_pe(p)
        _maybe(drain_prev, lambda: drain_out(pn))
        _maybe(issue_next, lambda: (issue_in(g + 1, pn), issue_pe(g + 1, pn)))
        compute(p)
        issue_out(g, p)

    issue_in(0, 0)
    issue_pe(0, 0)

    def tri_body(t, _):
        g = t * 3
        chunk(g, 0, t > 0, True)
        chunk(g + 1, 1, t > 0, True)
        chunk(g + 2, 2, True, True)
        return 0

    lax.fori_loop(0, NTRI, tri_body, 0)
    chunk(NCH - 2, 0, True, True)
    chunk(NCH - 1, 1, True, False)
    drain_out(0)
    drain_out(1)


def kernel(x, pe_table):
    mesh = plsc.VectorSubcoreMesh(core_axis_name="c", subcore_axis_name="s")
    run = pl.kernel(
        _body,
        mesh=mesh,
        out_type=jax.ShapeDtypeStruct((B, S, D), jnp.float32),
        scratch_types=[
            pltpu.VMEM((3, B, R, D), jnp.float32),
            pltpu.VMEM((3, R, D), jnp.float32),
            pltpu.SemaphoreType.DMA,
            pltpu.SemaphoreType.DMA,
            pltpu.SemaphoreType.DMA,
        ],
        compiler_params=pltpu.CompilerParams(use_tc_tiling_on_sc=True),
    )
    return run(x, pe_table)
